# baseline (speedup 1.0000x reference)
import functools
import math

import jax
import jax.numpy as jnp
import numpy as np
from jax import lax
from jax.experimental import pallas as pl
from jax.experimental.pallas import tpu as pltpu
from jax.experimental.pallas import tpu_sc as plsc

D = 512
B = 4
S = 2048
NFLAT = B * S
SCALE = math.sqrt(D)

_info = plsc.get_sparse_core_info()
NC, NS, L = _info.num_cores, _info.num_subcores, _info.num_lanes
NW = NC * NS
ROWS_PER_W = NFLAT // NW
CHUNK = 64
NCHUNK = ROWS_PER_W // CHUNK
WPS = S // ROWS_PER_W


def _positional_encoding() -> np.ndarray:
    position = np.arange(S, dtype=np.float32)[:, None]
    div_term = np.exp(
        np.arange(0, D, 2, dtype=np.float32) * (-math.log(10000.0) / D)
    )
    pe = np.zeros((S, D), dtype=np.float32)
    pe[:, 0::2] = np.sin(position * div_term)
    pe[:, 1::2] = np.cos(position * div_term)
    return pe


_PE = _positional_encoding()


def _make_kernel():
    mesh = plsc.VectorSubcoreMesh(core_axis_name="c", subcore_axis_name="s")

    @functools.partial(
        pl.kernel,
        mesh=mesh,
        out_type=jax.ShapeDtypeStruct((NFLAT, D), jnp.float32),
        scratch_types=[
            pltpu.VMEM((NCHUNK, CHUNK), jnp.int32),
            pltpu.VMEM((CHUNK, D), jnp.float32),
            pltpu.VMEM((CHUNK, D), jnp.float32),
            pltpu.SemaphoreType.DMA,
        ],
    )
    def emb(x_hbm, table_hbm, pe_hbm, out_hbm, idx_v, rows_v, pe_v, sem):
        wid = lax.axis_index("s") * NC + lax.axis_index("c")
        base = wid * ROWS_PER_W
        pe_base = lax.rem(wid, WPS) * ROWS_PER_W
        pltpu.sync_copy(x_hbm.at[wid], idx_v)

        def chunk(j, carry):
            row0 = base + j * CHUNK
            pltpu.async_copy(table_hbm.at[idx_v.at[j]], rows_v, sem).wait()
            pltpu.sync_copy(pe_hbm.at[pl.ds(pe_base + j * CHUNK, CHUNK)], pe_v)

            def row(r, carry2):
                for c in range(D // L):
                    sl = pl.ds(c * L, L)
                    rows_v[r, sl] = rows_v[r, sl] * SCALE + pe_v[r, sl]
                return carry2

            lax.fori_loop(0, CHUNK, row, 0)
            pltpu.sync_copy(rows_v, out_hbm.at[pl.ds(row0, CHUNK)])
            return carry

        lax.fori_loop(0, NCHUNK, chunk, 0)

    return emb


_emb = _make_kernel()


def kernel(x, table):
    x_r = x.reshape(NW, NCHUNK, CHUNK)
    pe = jnp.asarray(_PE)
    out = _emb(x_r, table, pe)
    return out.reshape(B, S, D)


# speedup vs baseline: 1.3084x; 1.3084x over previous
import functools
import math

import jax
import jax.numpy as jnp
import numpy as np
from jax import lax
from jax.experimental import pallas as pl
from jax.experimental.pallas import tpu as pltpu
from jax.experimental.pallas import tpu_sc as plsc

D = 512
B = 4
S = 2048
NFLAT = B * S
SCALE = math.sqrt(D)

NC, NS, L = 2, 16, 16
NW = NC * NS
POS_PER_W = S // NW


def _positional_encoding() -> np.ndarray:
    position = np.arange(S, dtype=np.float32)[:, None]
    div_term = np.exp(
        np.arange(0, D, 2, dtype=np.float32) * (-math.log(10000.0) / D)
    )
    pe = np.zeros((S, D), dtype=np.float32)
    pe[:, 0::2] = np.sin(position * div_term)
    pe[:, 1::2] = np.cos(position * div_term)
    return pe


_PE = _positional_encoding()


def _make_kernel():
    mesh = plsc.VectorSubcoreMesh(core_axis_name="c", subcore_axis_name="s")

    @functools.partial(
        pl.kernel,
        mesh=mesh,
        out_type=jax.ShapeDtypeStruct((NFLAT, D), jnp.float32),
        scratch_types=[
            pltpu.VMEM((B, POS_PER_W), jnp.int32),
            pltpu.VMEM((POS_PER_W, D), jnp.float32),
            pltpu.VMEM((POS_PER_W, D), jnp.float32),
            pltpu.VMEM((POS_PER_W, D), jnp.float32),
            pltpu.SemaphoreType.DMA,
            pltpu.SemaphoreType.DMA,
            pltpu.SemaphoreType.DMA,
            pltpu.SemaphoreType.DMA,
        ],
    )
    def emb(x_hbm, table_hbm, pe_hbm, out_hbm,
            idx_v, pe_v, rows0, rows1, g0, g1, o0, o1):
        wid = lax.axis_index("s") * NC + lax.axis_index("c")
        pos0 = wid * POS_PER_W
        pltpu.sync_copy(x_hbm.at[wid], idx_v)
        pltpu.sync_copy(pe_hbm.at[pl.ds(pos0, POS_PER_W)], pe_v)

        rows = (rows0, rows1)
        gsem = (g0, g1)
        osem = (o0, o1)
        g_h = [None, None]
        o_h = [None, None]
        g_h[0] = pltpu.async_copy(table_hbm.at[idx_v.at[0]], rows0, g0)
        for b in range(B):
            cur, nxt = b % 2, (b + 1) % 2
            if b + 1 < B:
                if o_h[nxt] is not None:
                    o_h[nxt].wait()
                g_h[nxt] = pltpu.async_copy(
                    table_hbm.at[idx_v.at[b + 1]], rows[nxt], gsem[nxt])
            g_h[cur].wait()

            def row(r, carry, cur=cur):
                for c in range(D // L):
                    sl = pl.ds(c * L, L)
                    rows[cur][r, sl] = rows[cur][r, sl] * SCALE + pe_v[r, sl]
                return carry

            lax.fori_loop(0, POS_PER_W, row, 0)
            o_h[cur] = pltpu.async_copy(
                rows[cur], out_hbm.at[pl.ds(b * S + pos0, POS_PER_W)],
                osem[cur])
        o_h[0].wait()
        o_h[1].wait()

    return emb


_emb = _make_kernel()


def kernel(x, table):
    x_r = x.reshape(B, NW, POS_PER_W).transpose(1, 0, 2)
    pe = jnp.asarray(_PE)
    out = _emb(x_r, table, pe)
    return out.reshape(B, S, D)


# speedup vs baseline: 1.3193x; 1.0083x over previous
import functools
import math

import jax
import jax.numpy as jnp
import numpy as np
from jax import lax
from jax.experimental import pallas as pl
from jax.experimental.pallas import tpu as pltpu
from jax.experimental.pallas import tpu_sc as plsc

D = 512
B = 4
S = 2048
NFLAT = B * S
SCALE = math.sqrt(D)

NC, NS, L = 2, 16, 16
NW = NC * NS
POS_PER_W = S // NW


def _positional_encoding() -> np.ndarray:
    position = np.arange(S, dtype=np.float32)[:, None]
    div_term = np.exp(
        np.arange(0, D, 2, dtype=np.float32) * (-math.log(10000.0) / D)
    )
    pe = np.zeros((S, D), dtype=np.float32)
    pe[:, 0::2] = np.sin(position * div_term)
    pe[:, 1::2] = np.cos(position * div_term)
    return pe


def _pe_bf16_interleaved() -> np.ndarray:
    pe = _positional_encoding().astype(jnp.bfloat16)
    return (
        pe.reshape(S, D // 32, 2, 16).transpose(0, 1, 3, 2).reshape(S * D)
    )


_PE_F32 = _positional_encoding()


def _make_kernel():
    mesh = plsc.VectorSubcoreMesh(core_axis_name="c", subcore_axis_name="s")

    @functools.partial(
        pl.kernel,
        mesh=mesh,
        out_type=jax.ShapeDtypeStruct((NFLAT, D), jnp.float32),
        scratch_types=[
            pltpu.VMEM((B, POS_PER_W), jnp.int32),
            pltpu.VMEM((POS_PER_W, D), jnp.float32),
            pltpu.VMEM((POS_PER_W, D), jnp.float32),
            pltpu.VMEM((POS_PER_W, D), jnp.float32),
            pltpu.SemaphoreType.DMA,
            pltpu.SemaphoreType.DMA,
            pltpu.SemaphoreType.DMA,
            pltpu.SemaphoreType.DMA,
        ],
    )
    def emb(x_hbm, table_hbm, pe_hbm, out_hbm,
            idx_v, pe_v, rows0, rows1, g0, g1, o0, o1):
        wid = lax.axis_index("s") * NC + lax.axis_index("c")
        pos0 = wid * POS_PER_W
        for b in range(B):
            pltpu.sync_copy(x_hbm.at[b, pl.ds(pos0, POS_PER_W)], idx_v.at[b])
        pltpu.sync_copy(pe_hbm.at[pl.ds(pos0, POS_PER_W)], pe_v)

        rows = (rows0, rows1)
        gsem = (g0, g1)
        osem = (o0, o1)
        g_h = [None, None]
        o_h = [None, None]
        g_h[0] = pltpu.async_copy(table_hbm.at[idx_v.at[0]], rows0, g0)
        for b in range(B):
            cur, nxt = b % 2, (b + 1) % 2
            if b + 1 < B:
                if o_h[nxt] is not None:
                    o_h[nxt].wait()
                g_h[nxt] = pltpu.async_copy(
                    table_hbm.at[idx_v.at[b + 1]], rows[nxt], gsem[nxt])
            g_h[cur].wait()

            def row(r, carry, cur=cur):
                for c in range(D // L):
                    sl = pl.ds(c * L, L)
                    rows[cur][r, sl] = rows[cur][r, sl] * SCALE + pe_v[r, sl]
                return carry

            lax.fori_loop(0, POS_PER_W, row, 0)
            o_h[cur] = pltpu.async_copy(
                rows[cur], out_hbm.at[pl.ds(b * S + pos0, POS_PER_W)],
                osem[cur])
        o_h[0].wait()
        o_h[1].wait()

    return emb


_emb = _make_kernel()


def kernel(x, table):
    pe = jnp.asarray(_PE_F32)
    out = _emb(x, table, pe)
    return out.reshape(B, S, D)
